# baseline (speedup 1.0000x reference)
import functools
import jax
import jax.numpy as jnp
from jax import lax
from jax.experimental import pallas as pl
from jax.experimental.pallas import tpu as pltpu
from jax.experimental.pallas import tpu_sc as plsc

H, W = 64, 220
A = 9
N = H * W * A
ROWS, LANES = 990, 128
G = 32
B = 4
STRIDE = 8.0
FG_T, IGN_T = 0.5, 0.4
STDS = (0.1, 0.1, 0.2, 0.2)
LN2 = 0.6931471805599453
SQRT2 = 1.4142135623730951


def _tc_body(cls_ref, box_ref, gts_ref, lbl_ref, anc_ref, sums_ref, binds_ref):
    i = pl.program_id(0)

    r = lax.broadcasted_iota(jnp.int32, (ROWS, LANES), 0)
    l = lax.broadcasted_iota(jnp.int32, (ROWS, LANES), 1)
    n = r * LANES + l
    a = n % A
    cell = n // A
    wi = cell % W
    hi = cell // W
    gx = wi.astype(jnp.float32) * STRIDE
    gy = hi.astype(jnp.float32) * STRIDE

    ax1 = jnp.zeros((ROWS, LANES), jnp.float32)
    ay1 = jnp.zeros((ROWS, LANES), jnp.float32)
    ax2 = jnp.zeros((ROWS, LANES), jnp.float32)
    ay2 = jnp.zeros((ROWS, LANES), jnp.float32)
    for k in range(A):
        sel = a == k
        ax1 = jnp.where(sel, anc_ref[k, 0], ax1)
        ay1 = jnp.where(sel, anc_ref[k, 1], ay1)
        ax2 = jnp.where(sel, anc_ref[k, 2], ax2)
        ay2 = jnp.where(sel, anc_ref[k, 3], ay2)

    x1 = gx + ax1
    y1 = gy + ay1
    x2 = gx + ax2
    y2 = gy + ay2
    aa = (x2 - x1) * (y2 - y1)
    rw = x2 - x1 + 1.0
    rh = y2 - y1 + 1.0
    rcx = x1 + 0.5 * rw
    rcy = y1 + 0.5 * rh

    BIG = jnp.int32(1 << 30)
    best = None
    for g in range(G):
        gx1 = gts_ref[i, g, 0]
        gy1 = gts_ref[i, g, 1]
        gx2 = gts_ref[i, g, 2]
        gy2 = gts_ref[i, g, 3]
        lblg = lbl_ref[i, g]
        ab = (gx2 - gx1) * (gy2 - gy1)
        iw = jnp.maximum(jnp.minimum(x2, gx2) - jnp.maximum(x1, gx1), 0.0)
        ih = jnp.maximum(jnp.minimum(y2, gy2) - jnp.maximum(y1, gy1), 0.0)
        inter = iw * ih
        iou = inter / jnp.maximum(aa + ab - inter, 1e-8)
        mg = jnp.max(iou)
        binds_ref[0, i * G + g] = jnp.min(jnp.where(iou >= mg, n, BIG))
        if g == 0:
            best = iou
            labv = jnp.full((ROWS, LANES), lblg, jnp.int32)
            mx1 = jnp.full((ROWS, LANES), gx1, jnp.float32)
            my1 = jnp.full((ROWS, LANES), gy1, jnp.float32)
            mx2 = jnp.full((ROWS, LANES), gx2, jnp.float32)
            my2 = jnp.full((ROWS, LANES), gy2, jnp.float32)
        else:
            upd = iou > best
            best = jnp.where(upd, iou, best)
            labv = jnp.where(upd, lblg, labv)
            mx1 = jnp.where(upd, gx1, mx1)
            my1 = jnp.where(upd, gy1, my1)
            mx2 = jnp.where(upd, gx2, mx2)
            my2 = jnp.where(upd, gy2, my2)

    fg = best >= FG_T
    ign = (best >= IGN_T) & (~fg)
    wv = jnp.where(ign, 0.0, 1.0)
    labels = jnp.where(fg, labv, 0)

    c0 = cls_ref[0, 0]
    c1 = cls_ref[0, 1]
    c2 = cls_ref[0, 2]
    c3 = cls_ref[0, 3]
    m = jnp.maximum(jnp.maximum(c0, c1), jnp.maximum(c2, c3))
    lse = jnp.log(jnp.exp(c0 - m) + jnp.exp(c1 - m)
                  + jnp.exp(c2 - m) + jnp.exp(c3 - m)) + m
    csel = jnp.where(labels == 0, c0,
                     jnp.where(labels == 1, c1,
                               jnp.where(labels == 2, c2, c3)))
    ce = lse - csel

    gw = mx2 - mx1 + 1.0
    gh = my2 - my1 + 1.0
    gcx = mx1 + 0.5 * gw
    gcy = my1 + 0.5 * gh
    b0 = box_ref[0, 0]
    b1 = box_ref[0, 1]
    b2 = box_ref[0, 2]
    b3 = box_ref[0, 3]
    t0 = ((gcx - rcx) / rw) / STDS[0]
    t1 = ((gcy - rcy) / rh) / STDS[1]
    t2 = jnp.log(gw / rw) / STDS[2]
    t3 = jnp.log(gh / rh) / STDS[3]
    sl1 = jnp.zeros((ROWS, LANES), jnp.float32)
    for bv, tv in ((b0, t0), (b1, t1), (b2, t2), (b3, t3)):
        d = bv - tv
        ad = jnp.abs(d)
        sl1 = sl1 + jnp.where(ad < 1.0, 0.5 * d * d, ad - 0.5)
    fgf = fg.astype(jnp.float32)

    d0 = b0 * STDS[0]
    d1 = b1 * STDS[1]
    d2 = b2 * STDS[2]
    d3 = b3 * STDS[3]
    pcx = d0 * rw + rcx
    pcy = d1 * rh + rcy
    pw = jnp.exp(jnp.clip(d2, -4.0, 4.0)) * rw
    ph = jnp.exp(jnp.clip(d3, -4.0, 4.0)) * rh
    px1 = pcx - 0.5 * pw
    py1 = pcy - 0.5 * ph
    px2 = pcx + 0.5 * pw
    py2 = pcy + 0.5 * ph
    iw = jnp.maximum(jnp.minimum(px2, mx2) - jnp.maximum(px1, mx1), 0.0)
    ih = jnp.maximum(jnp.minimum(py2, my2) - jnp.maximum(py1, my1), 0.0)
    inter = iw * ih
    pa = (px2 - px1) * (py2 - py1)
    ga = (mx2 - mx1) * (my2 - my1)
    ious = inter / jnp.maximum(pa + ga - inter, 1e-8)

    sums_ref[0, i * 16 + 0] = jnp.sum(ce * wv)
    sums_ref[0, i * 16 + 1] = jnp.sum(wv)
    sums_ref[0, i * 16 + 2] = jnp.sum(sl1 * fgf)
    sums_ref[0, i * 16 + 3] = jnp.sum((1.0 - ious) * fgf)
    sums_ref[0, i * 16 + 4] = jnp.sum(fgf)


def _tc_stage(cls_t, box_t, gts, gt_labels, anchors):
    return pl.pallas_call(
        _tc_body,
        grid=(B,),
        in_specs=[
            pl.BlockSpec((1, 4, ROWS, LANES), lambda i: (i, 0, 0, 0)),
            pl.BlockSpec((1, 4, ROWS, LANES), lambda i: (i, 0, 0, 0)),
            pl.BlockSpec(memory_space=pltpu.SMEM),
            pl.BlockSpec(memory_space=pltpu.SMEM),
            pl.BlockSpec(memory_space=pltpu.SMEM),
        ],
        out_specs=[
            pl.BlockSpec((1, 16 * B), lambda i: (0, 0), memory_space=pltpu.SMEM),
            pl.BlockSpec((1, G * B), lambda i: (0, 0), memory_space=pltpu.SMEM),
        ],
        out_shape=[
            jax.ShapeDtypeStruct((1, 16 * B), jnp.float32),
            jax.ShapeDtypeStruct((1, G * B), jnp.int32),
        ],
    )(cls_t, box_t, gts, gt_labels, anchors)


def _sc_log(x):
    ix = lax.bitcast_convert_type(x, jnp.int32)
    e = ((ix >> 23) & 0xFF) - 127
    mb = (ix & 0x007FFFFF) | 0x3F800000
    mv = lax.bitcast_convert_type(mb, jnp.float32)
    adj = mv > SQRT2
    mv = jnp.where(adj, mv * 0.5, mv)
    e = (e + jnp.where(adj, 1, 0)).astype(jnp.float32)
    z = (mv - 1.0) / (mv + 1.0)
    z2 = z * z
    p = z * (2.0 + z2 * (2.0 / 3.0 + z2 * (2.0 / 5.0
             + z2 * (2.0 / 7.0 + z2 * (2.0 / 9.0 + z2 * (2.0 / 11.0))))))
    return e * LN2 + p


def _sc_body(cls_h, box_h, gts_h, lbl_h, anc_h, sums_h, binds_h, out_h,
             gts_v, lbl_v, binds_v, sums_v, anc_v, idx_c, idx_b,
             gcls, gbox, outs, sem1, sem2):
    wid = lax.axis_index("c") * 16 + lax.axis_index("s")

    @pl.when(wid < B)
    def _():
        i = wid
        pltpu.sync_copy(gts_h.at[pl.ds(i * 4 * G, 4 * G)], gts_v)
        pltpu.sync_copy(lbl_h.at[pl.ds(i * G, G)], lbl_v)
        pltpu.sync_copy(binds_h.at[pl.ds(i * G, G)], binds_v)
        pltpu.sync_copy(sums_h.at[pl.ds(i * 16, 16)], sums_v)
        pltpu.sync_copy(anc_h, anc_v)

        gvv = [gts_v[pl.ds(k * 16, 16)] for k in range(8)]
        lvv = [lbl_v[pl.ds(k * 16, 16)] for k in range(2)]
        bvv = [binds_v[pl.ds(k * 16, 16)] for k in range(2)]
        avv = [anc_v[pl.ds(k * 16, 16)] for k in range(3)]
        svv = sums_v[pl.ds(0, 16)]
        iot = lax.iota(jnp.int32, 16)

        def _g16(x, idx):
            dn = lax.GatherDimensionNumbers(offset_dims=(),
                                            collapsed_slice_dims=(0,),
                                            start_index_map=(0,))
            return lax.gather(x, idx[:, None], dn, (1,),
                              mode=lax.GatherScatterMode.PROMISE_IN_BOUNDS)

        def _splat(v, k):
            return _g16(v, jnp.full((16,), k, jnp.int32))

        def _lane_sum(x):
            for sh in (8, 4, 2, 1):
                x = x + _g16(x, (iot + sh) & 15)
            return x

        def gsc(idx):
            return _splat(gvv[idx // 16], idx % 16)

        def _divmod(x, d):
            q = (x.astype(jnp.float32) * (1.0 / d)).astype(jnp.int32)
            r = x - q * d
            q = q + jnp.where(r >= d, 1, 0) - jnp.where(r < 0, 1, 0)
            r = x - q * d
            return q, r

        def _gat(vregs, idx):
            r = _g16(vregs[0], idx & 15)
            for k in range(1, len(vregs)):
                r = jnp.where((idx >> 4) == k, _g16(vregs[k], idx & 15), r)
            return r

        base = i * N * 4
        for h in range(2):
            nj = bvv[h]
            for c in range(4):
                idx_c[pl.ds(c * G + h * 16, 16)] = base + nj * 4 + c
                idx_b[pl.ds(c * G + h * 16, 16)] = base + nj * 4 + c
        cp1 = pltpu.async_copy(cls_h.at[idx_c], gcls, sem1)
        cp2 = pltpu.async_copy(box_h.at[idx_b], gbox, sem2)
        cp1.wait()
        cp2.wait()

        dd = [jnp.zeros((16,), jnp.float32) for _ in range(5)]
        for h in range(2):
            nj = bvv[h]
            cell, a = _divmod(nj, A)
            hi, wi = _divmod(cell, W)
            gxv = wi.astype(jnp.float32) * STRIDE
            gyv = hi.astype(jnp.float32) * STRIDE
            ax1 = jnp.zeros((16,), jnp.float32)
            ay1 = jnp.zeros((16,), jnp.float32)
            ax2 = jnp.zeros((16,), jnp.float32)
            ay2 = jnp.zeros((16,), jnp.float32)
            for k in range(A):
                sel = a == k
                ax1 = jnp.where(sel, _splat(avv[(k * 4 + 0) // 16], (k * 4 + 0) % 16), ax1)
                ay1 = jnp.where(sel, _splat(avv[(k * 4 + 1) // 16], (k * 4 + 1) % 16), ay1)
                ax2 = jnp.where(sel, _splat(avv[(k * 4 + 2) // 16], (k * 4 + 2) % 16), ax2)
                ay2 = jnp.where(sel, _splat(avv[(k * 4 + 3) // 16], (k * 4 + 3) % 16), ay2)
            x1 = gxv + ax1
            y1 = gyv + ay1
            x2 = gxv + ax2
            y2 = gyv + ay2
            aa = (x2 - x1) * (y2 - y1)
            rw = x2 - x1 + 1.0
            rh = y2 - y1 + 1.0
            rcx = x1 + 0.5 * rw
            rcy = y1 + 0.5 * rh

            best = None
            for g in range(G):
                g1 = gsc(g * 4 + 0)
                g2 = gsc(g * 4 + 1)
                g3 = gsc(g * 4 + 2)
                g4 = gsc(g * 4 + 3)
                lblg = _splat(lvv[g // 16], g % 16)
                ab = (g3 - g1) * (g4 - g2)
                iw = jnp.maximum(jnp.minimum(x2, g3) - jnp.maximum(x1, g1), 0.0)
                ih = jnp.maximum(jnp.minimum(y2, g4) - jnp.maximum(y1, g2), 0.0)
                inter = iw * ih
                iou = inter / jnp.maximum(aa + ab - inter, 1e-8)
                if g == 0:
                    best = iou
                    labv = lblg
                    ox1 = g1
                    oy1 = g2
                    ox2 = g3
                    oy2 = g4
                else:
                    upd = iou > best
                    best = jnp.where(upd, iou, best)
                    labv = jnp.where(upd, lblg, labv)
                    ox1 = jnp.where(upd, g1, ox1)
                    oy1 = jnp.where(upd, g2, oy1)
                    ox2 = jnp.where(upd, g3, ox2)
                    oy2 = jnp.where(upd, g4, oy2)
            fg0f = jnp.where(best >= FG_T, 1.0, 0.0)
            w0 = jnp.where(best >= IGN_T, fg0f, 1.0)
            lab_old = jnp.where(best >= FG_T, labv, 0)

            mm = jnp.zeros((16,), jnp.int32)
            occ = jnp.full((16,), G, jnp.int32)
            for gp in range(G):
                cond = nj == _splat(bvv[gp // 16], gp % 16)
                mm = jnp.where(cond, gp, mm)
                occ = jnp.minimum(occ, jnp.where(cond, gp, G))
            j_idx = iot + h * 16
            first = occ == j_idx

            ngx1 = _gat(gvv, mm * 4 + 0)
            ngy1 = _gat(gvv, mm * 4 + 1)
            ngx2 = _gat(gvv, mm * 4 + 2)
            ngy2 = _gat(gvv, mm * 4 + 3)
            lab_new = _gat(lvv, mm)

            c0 = gcls[pl.ds(0 * G + h * 16, 16)]
            c1 = gcls[pl.ds(1 * G + h * 16, 16)]
            c2 = gcls[pl.ds(2 * G + h * 16, 16)]
            c3 = gcls[pl.ds(3 * G + h * 16, 16)]
            b0 = gbox[pl.ds(0 * G + h * 16, 16)]
            b1 = gbox[pl.ds(1 * G + h * 16, 16)]
            b2 = gbox[pl.ds(2 * G + h * 16, 16)]
            b3 = gbox[pl.ds(3 * G + h * 16, 16)]

            mx = jnp.maximum(jnp.maximum(c0, c1), jnp.maximum(c2, c3))
            lse = _sc_log(jnp.exp(c0 - mx) + jnp.exp(c1 - mx)
                          + jnp.exp(c2 - mx) + jnp.exp(c3 - mx)) + mx

            def csel(lab):
                return jnp.where(lab == 0, c0,
                                 jnp.where(lab == 1, c1,
                                           jnp.where(lab == 2, c2, c3)))
            ce_old = lse - csel(lab_old)
            ce_new = lse - csel(lab_new)

            def sl1_of(q1, q2, q3, q4):
                gwv = q3 - q1 + 1.0
                ghv = q4 - q2 + 1.0
                gcx = q1 + 0.5 * gwv
                gcy = q2 + 0.5 * ghv
                t0 = ((gcx - rcx) / rw) / STDS[0]
                t1 = ((gcy - rcy) / rh) / STDS[1]
                t2 = _sc_log(gwv / rw) / STDS[2]
                t3 = _sc_log(ghv / rh) / STDS[3]
                s = jnp.zeros((16,), jnp.float32)
                for bv, tv in ((b0, t0), (b1, t1), (b2, t2), (b3, t3)):
                    d = bv - tv
                    ad = jnp.abs(d)
                    s = s + jnp.where(ad < 1.0, 0.5 * d * d, ad - 0.5)
                return s

            sl1_old = sl1_of(ox1, oy1, ox2, oy2)
            sl1_new = sl1_of(ngx1, ngy1, ngx2, ngy2)

            d0 = b0 * STDS[0]
            d1 = b1 * STDS[1]
            d2 = b2 * STDS[2]
            d3 = b3 * STDS[3]
            pcx = d0 * rw + rcx
            pcy = d1 * rh + rcy
            pw = jnp.exp(jnp.clip(d2, -4.0, 4.0)) * rw
            ph = jnp.exp(jnp.clip(d3, -4.0, 4.0)) * rh
            px1 = pcx - 0.5 * pw
            py1 = pcy - 0.5 * ph
            px2 = pcx + 0.5 * pw
            py2 = pcy + 0.5 * ph
            pa = (px2 - px1) * (py2 - py1)

            def iou_of(q1, q2, q3, q4):
                iw = jnp.maximum(jnp.minimum(px2, q3) - jnp.maximum(px1, q1), 0.0)
                ih = jnp.maximum(jnp.minimum(py2, q4) - jnp.maximum(py1, q2), 0.0)
                inter = iw * ih
                ga = (q3 - q1) * (q4 - q2)
                return inter / jnp.maximum(pa + ga - inter, 1e-8)

            iou_old = iou_of(ox1, oy1, ox2, oy2)
            iou_new = iou_of(ngx1, ngy1, ngx2, ngy2)

            zf = jnp.zeros((16,), jnp.float32)
            dd[0] = dd[0] + jnp.where(first, ce_new - ce_old * w0, zf)
            dd[1] = dd[1] + jnp.where(first, 1.0 - w0, zf)
            dd[2] = dd[2] + jnp.where(first, sl1_new - sl1_old * fg0f, zf)
            dd[3] = dd[3] + jnp.where(first,
                                      (1.0 - iou_new) - (1.0 - iou_old) * fg0f,
                                      zf)
            dd[4] = dd[4] + jnp.where(first, 1.0 - fg0f, zf)

        cls_num = _splat(svv, 0) + _lane_sum(dd[0])
        w_sum = _splat(svv, 1) + _lane_sum(dd[1])
        bb_num = _splat(svv, 2) + _lane_sum(dd[2])
        iou_num = _splat(svv, 3) + _lane_sum(dd[3])
        nfg = _splat(svv, 4) + _lane_sum(dd[4])
        loss = (cls_num / jnp.maximum(w_sum, 1.0)
                + (bb_num + iou_num) / jnp.maximum(nfg, 1.0))
        outs[...] = loss
        pltpu.sync_copy(outs, out_h.at[pl.ds(i * 16, 16)])


def _sc_stage(clsf, boxf, gtsf, lblf, ancf, sumsf, bindsf):
    mesh = plsc.VectorSubcoreMesh(core_axis_name="c", subcore_axis_name="s")
    run = functools.partial(
        pl.kernel,
        mesh=mesh,
        out_type=jax.ShapeDtypeStruct((B * 16,), jnp.float32),
        scratch_types=[
            pltpu.VMEM((4 * G,), jnp.float32),
            pltpu.VMEM((G,), jnp.int32),
            pltpu.VMEM((G,), jnp.int32),
            pltpu.VMEM((16,), jnp.float32),
            pltpu.VMEM((48,), jnp.float32),
            pltpu.VMEM((128,), jnp.int32),
            pltpu.VMEM((128,), jnp.int32),
            pltpu.VMEM((128,), jnp.float32),
            pltpu.VMEM((128,), jnp.float32),
            pltpu.VMEM((16,), jnp.float32),
            pltpu.SemaphoreType.DMA,
            pltpu.SemaphoreType.DMA,
        ],
    )(_sc_body)
    return run(clsf, boxf, gtsf, lblf, ancf, sumsf, bindsf)


@jax.jit
def kernel(cls, bbox_2d, gts, anchors, gt_labels):
    cls_t = cls.transpose(0, 2, 1).reshape(B, 4, ROWS, LANES)
    box_t = bbox_2d.transpose(0, 2, 1).reshape(B, 4, ROWS, LANES)
    lbl = gt_labels.astype(jnp.int32)
    sums, binds = _tc_stage(cls_t, box_t, gts, lbl, anchors)
    out = _sc_stage(
        cls.reshape(-1),
        bbox_2d.reshape(-1),
        gts.reshape(-1),
        lbl.reshape(-1),
        jnp.pad(anchors.reshape(-1), (0, 12)),
        sums.reshape(-1),
        binds.reshape(-1),
    )
    return jnp.mean(out.reshape(B, 16)[:, 0])


# speedup vs baseline: 7.2331x; 7.2331x over previous
import functools
import jax
import jax.numpy as jnp
from jax import lax
from jax.experimental import pallas as pl
from jax.experimental.pallas import tpu as pltpu
from jax.experimental.pallas import tpu_sc as plsc

H, W = 64, 220
A = 9
N = H * W * A
ROWS, LANES = 990, 128
G = 32
B = 4
STRIDE = 8.0
FG_T, IGN_T = 0.5, 0.4
STDS = (0.1, 0.1, 0.2, 0.2)
LN2 = 0.6931471805599453
SQRT2 = 1.4142135623730951


def _tc_body(cls_ref, box_ref, gts_ref, lbl_ref, anc_ref, sums_ref, binds_ref, gath_ref):
    i = pl.program_id(0)

    r = lax.broadcasted_iota(jnp.int32, (ROWS, LANES), 0)
    l = lax.broadcasted_iota(jnp.int32, (ROWS, LANES), 1)
    n = r * LANES + l
    a = n % A
    cell = n // A
    wi = cell % W
    hi = cell // W
    gx = wi.astype(jnp.float32) * STRIDE
    gy = hi.astype(jnp.float32) * STRIDE

    ax1 = jnp.zeros((ROWS, LANES), jnp.float32)
    ay1 = jnp.zeros((ROWS, LANES), jnp.float32)
    ax2 = jnp.zeros((ROWS, LANES), jnp.float32)
    ay2 = jnp.zeros((ROWS, LANES), jnp.float32)
    for k in range(A):
        sel = a == k
        ax1 = jnp.where(sel, anc_ref[k, 0], ax1)
        ay1 = jnp.where(sel, anc_ref[k, 1], ay1)
        ax2 = jnp.where(sel, anc_ref[k, 2], ax2)
        ay2 = jnp.where(sel, anc_ref[k, 3], ay2)

    x1 = gx + ax1
    y1 = gy + ay1
    x2 = gx + ax2
    y2 = gy + ay2
    aa = (x2 - x1) * (y2 - y1)
    rw = x2 - x1 + 1.0
    rh = y2 - y1 + 1.0
    rcx = x1 + 0.5 * rw
    rcy = y1 + 0.5 * rh

    BIG = jnp.int32(1 << 30)
    best = None
    bis = []
    for g in range(G):
        gx1 = gts_ref[i, g, 0]
        gy1 = gts_ref[i, g, 1]
        gx2 = gts_ref[i, g, 2]
        gy2 = gts_ref[i, g, 3]
        lblg = lbl_ref[i, g]
        ab = (gx2 - gx1) * (gy2 - gy1)
        iw = jnp.maximum(jnp.minimum(x2, gx2) - jnp.maximum(x1, gx1), 0.0)
        ih = jnp.maximum(jnp.minimum(y2, gy2) - jnp.maximum(y1, gy1), 0.0)
        inter = iw * ih
        iou = inter / jnp.maximum(aa + ab - inter, 1e-8)
        mg = jnp.max(iou)
        bi = jnp.min(jnp.where(iou >= mg, n, BIG))
        binds_ref[0, i * G + g] = bi
        bis.append(bi)
        if g == 0:
            best = iou
            labv = jnp.full((ROWS, LANES), lblg, jnp.int32)
            mx1 = jnp.full((ROWS, LANES), gx1, jnp.float32)
            my1 = jnp.full((ROWS, LANES), gy1, jnp.float32)
            mx2 = jnp.full((ROWS, LANES), gx2, jnp.float32)
            my2 = jnp.full((ROWS, LANES), gy2, jnp.float32)
        else:
            upd = iou > best
            best = jnp.where(upd, iou, best)
            labv = jnp.where(upd, lblg, labv)
            mx1 = jnp.where(upd, gx1, mx1)
            my1 = jnp.where(upd, gy1, my1)
            mx2 = jnp.where(upd, gx2, mx2)
            my2 = jnp.where(upd, gy2, my2)

    liota = lax.broadcasted_iota(jnp.int32, (1, LANES), 1)
    for g in range(G):
        rr = bis[g] // LANES
        ll = bis[g] % LANES
        lm = liota == ll
        for c in range(4):
            row = cls_ref[0, c, pl.ds(rr, 1), :]
            gath_ref[0, i * 256 + c * G + g] = jnp.sum(jnp.where(lm, row, 0.0))
            row = box_ref[0, c, pl.ds(rr, 1), :]
            gath_ref[0, i * 256 + (c + 4) * G + g] = jnp.sum(
                jnp.where(lm, row, 0.0))

    fg = best >= FG_T
    ign = (best >= IGN_T) & (~fg)
    wv = jnp.where(ign, 0.0, 1.0)
    labels = jnp.where(fg, labv, 0)

    c0 = cls_ref[0, 0]
    c1 = cls_ref[0, 1]
    c2 = cls_ref[0, 2]
    c3 = cls_ref[0, 3]
    m = jnp.maximum(jnp.maximum(c0, c1), jnp.maximum(c2, c3))
    lse = jnp.log(jnp.exp(c0 - m) + jnp.exp(c1 - m)
                  + jnp.exp(c2 - m) + jnp.exp(c3 - m)) + m
    csel = jnp.where(labels == 0, c0,
                     jnp.where(labels == 1, c1,
                               jnp.where(labels == 2, c2, c3)))
    ce = lse - csel

    gw = mx2 - mx1 + 1.0
    gh = my2 - my1 + 1.0
    gcx = mx1 + 0.5 * gw
    gcy = my1 + 0.5 * gh
    b0 = box_ref[0, 0]
    b1 = box_ref[0, 1]
    b2 = box_ref[0, 2]
    b3 = box_ref[0, 3]
    t0 = ((gcx - rcx) / rw) / STDS[0]
    t1 = ((gcy - rcy) / rh) / STDS[1]
    t2 = jnp.log(gw / rw) / STDS[2]
    t3 = jnp.log(gh / rh) / STDS[3]
    sl1 = jnp.zeros((ROWS, LANES), jnp.float32)
    for bv, tv in ((b0, t0), (b1, t1), (b2, t2), (b3, t3)):
        d = bv - tv
        ad = jnp.abs(d)
        sl1 = sl1 + jnp.where(ad < 1.0, 0.5 * d * d, ad - 0.5)
    fgf = fg.astype(jnp.float32)

    d0 = b0 * STDS[0]
    d1 = b1 * STDS[1]
    d2 = b2 * STDS[2]
    d3 = b3 * STDS[3]
    pcx = d0 * rw + rcx
    pcy = d1 * rh + rcy
    pw = jnp.exp(jnp.clip(d2, -4.0, 4.0)) * rw
    ph = jnp.exp(jnp.clip(d3, -4.0, 4.0)) * rh
    px1 = pcx - 0.5 * pw
    py1 = pcy - 0.5 * ph
    px2 = pcx + 0.5 * pw
    py2 = pcy + 0.5 * ph
    iw = jnp.maximum(jnp.minimum(px2, mx2) - jnp.maximum(px1, mx1), 0.0)
    ih = jnp.maximum(jnp.minimum(py2, my2) - jnp.maximum(py1, my1), 0.0)
    inter = iw * ih
    pa = (px2 - px1) * (py2 - py1)
    ga = (mx2 - mx1) * (my2 - my1)
    ious = inter / jnp.maximum(pa + ga - inter, 1e-8)

    sums_ref[0, i * 16 + 0] = jnp.sum(ce * wv)
    sums_ref[0, i * 16 + 1] = jnp.sum(wv)
    sums_ref[0, i * 16 + 2] = jnp.sum(sl1 * fgf)
    sums_ref[0, i * 16 + 3] = jnp.sum((1.0 - ious) * fgf)
    sums_ref[0, i * 16 + 4] = jnp.sum(fgf)


def _tc_stage(cls_t, box_t, gts, gt_labels, anchors):
    return pl.pallas_call(
        _tc_body,
        grid=(B,),
        in_specs=[
            pl.BlockSpec((1, 4, ROWS, LANES), lambda i: (i, 0, 0, 0)),
            pl.BlockSpec((1, 4, ROWS, LANES), lambda i: (i, 0, 0, 0)),
            pl.BlockSpec(memory_space=pltpu.SMEM),
            pl.BlockSpec(memory_space=pltpu.SMEM),
            pl.BlockSpec(memory_space=pltpu.SMEM),
        ],
        out_specs=[
            pl.BlockSpec((1, 16 * B), lambda i: (0, 0), memory_space=pltpu.SMEM),
            pl.BlockSpec((1, G * B), lambda i: (0, 0), memory_space=pltpu.SMEM),
            pl.BlockSpec((1, 256 * B), lambda i: (0, 0),
                         memory_space=pltpu.SMEM),
        ],
        out_shape=[
            jax.ShapeDtypeStruct((1, 16 * B), jnp.float32),
            jax.ShapeDtypeStruct((1, G * B), jnp.int32),
            jax.ShapeDtypeStruct((1, 256 * B), jnp.float32),
        ],
    )(cls_t, box_t, gts, gt_labels, anchors)


def _sc_log(x):
    ix = lax.bitcast_convert_type(x, jnp.int32)
    e = ((ix >> 23) & 0xFF) - 127
    mb = (ix & 0x007FFFFF) | 0x3F800000
    mv = lax.bitcast_convert_type(mb, jnp.float32)
    adj = mv > SQRT2
    mv = jnp.where(adj, mv * 0.5, mv)
    e = (e + jnp.where(adj, 1, 0)).astype(jnp.float32)
    z = (mv - 1.0) / (mv + 1.0)
    z2 = z * z
    p = z * (2.0 + z2 * (2.0 / 3.0 + z2 * (2.0 / 5.0
             + z2 * (2.0 / 7.0 + z2 * (2.0 / 9.0 + z2 * (2.0 / 11.0))))))
    return e * LN2 + p


def _sc_body(gts_h, lbl_h, anc_h, sums_h, binds_h, gath_h, out_h,
             gts_v, lbl_v, binds_v, sums_v, anc_v, gall, outs):
    wid = lax.axis_index("c") * 16 + lax.axis_index("s")

    @pl.when(wid < B)
    def _():
        i = wid
        pltpu.sync_copy(gts_h.at[pl.ds(i * 4 * G, 4 * G)], gts_v)
        pltpu.sync_copy(lbl_h.at[pl.ds(i * G, G)], lbl_v)
        pltpu.sync_copy(binds_h.at[pl.ds(i * G, G)], binds_v)
        pltpu.sync_copy(sums_h.at[pl.ds(i * 16, 16)], sums_v)
        pltpu.sync_copy(anc_h, anc_v)
        pltpu.sync_copy(gath_h.at[pl.ds(i * 256, 256)], gall)

        gvv = [gts_v[pl.ds(k * 16, 16)] for k in range(8)]
        lvv = [lbl_v[pl.ds(k * 16, 16)] for k in range(2)]
        bvv = [binds_v[pl.ds(k * 16, 16)] for k in range(2)]
        avv = [anc_v[pl.ds(k * 16, 16)] for k in range(3)]
        svv = sums_v[pl.ds(0, 16)]
        iot = lax.iota(jnp.int32, 16)

        def _g16(x, idx):
            dn = lax.GatherDimensionNumbers(offset_dims=(),
                                            collapsed_slice_dims=(0,),
                                            start_index_map=(0,))
            return lax.gather(x, idx[:, None], dn, (1,),
                              mode=lax.GatherScatterMode.PROMISE_IN_BOUNDS)

        def _splat(v, k):
            return _g16(v, jnp.full((16,), k, jnp.int32))

        def _lane_sum(x):
            for sh in (8, 4, 2, 1):
                x = x + _g16(x, (iot + sh) & 15)
            return x

        def gsc(idx):
            return _splat(gvv[idx // 16], idx % 16)

        def _divmod(x, d):
            q = (x.astype(jnp.float32) * (1.0 / d)).astype(jnp.int32)
            r = x - q * d
            q = q + jnp.where(r >= d, 1, 0) - jnp.where(r < 0, 1, 0)
            r = x - q * d
            return q, r

        def _gat(vregs, idx):
            r = _g16(vregs[0], idx & 15)
            for k in range(1, len(vregs)):
                r = jnp.where((idx >> 4) == k, _g16(vregs[k], idx & 15), r)
            return r

        dd = [jnp.zeros((16,), jnp.float32) for _ in range(5)]
        for h in range(2):
            nj = bvv[h]
            cell, a = _divmod(nj, A)
            hi, wi = _divmod(cell, W)
            gxv = wi.astype(jnp.float32) * STRIDE
            gyv = hi.astype(jnp.float32) * STRIDE
            ax1 = jnp.zeros((16,), jnp.float32)
            ay1 = jnp.zeros((16,), jnp.float32)
            ax2 = jnp.zeros((16,), jnp.float32)
            ay2 = jnp.zeros((16,), jnp.float32)
            for k in range(A):
                sel = a == k
                ax1 = jnp.where(sel, _splat(avv[(k * 4 + 0) // 16], (k * 4 + 0) % 16), ax1)
                ay1 = jnp.where(sel, _splat(avv[(k * 4 + 1) // 16], (k * 4 + 1) % 16), ay1)
                ax2 = jnp.where(sel, _splat(avv[(k * 4 + 2) // 16], (k * 4 + 2) % 16), ax2)
                ay2 = jnp.where(sel, _splat(avv[(k * 4 + 3) // 16], (k * 4 + 3) % 16), ay2)
            x1 = gxv + ax1
            y1 = gyv + ay1
            x2 = gxv + ax2
            y2 = gyv + ay2
            aa = (x2 - x1) * (y2 - y1)
            rw = x2 - x1 + 1.0
            rh = y2 - y1 + 1.0
            rcx = x1 + 0.5 * rw
            rcy = y1 + 0.5 * rh

            best = None
            for g in range(G):
                g1 = gsc(g * 4 + 0)
                g2 = gsc(g * 4 + 1)
                g3 = gsc(g * 4 + 2)
                g4 = gsc(g * 4 + 3)
                lblg = _splat(lvv[g // 16], g % 16)
                ab = (g3 - g1) * (g4 - g2)
                iw = jnp.maximum(jnp.minimum(x2, g3) - jnp.maximum(x1, g1), 0.0)
                ih = jnp.maximum(jnp.minimum(y2, g4) - jnp.maximum(y1, g2), 0.0)
                inter = iw * ih
                iou = inter / jnp.maximum(aa + ab - inter, 1e-8)
                if g == 0:
                    best = iou
                    labv = lblg
                    ox1 = g1
                    oy1 = g2
                    ox2 = g3
                    oy2 = g4
                else:
                    upd = iou > best
                    best = jnp.where(upd, iou, best)
                    labv = jnp.where(upd, lblg, labv)
                    ox1 = jnp.where(upd, g1, ox1)
                    oy1 = jnp.where(upd, g2, oy1)
                    ox2 = jnp.where(upd, g3, ox2)
                    oy2 = jnp.where(upd, g4, oy2)
            fg0f = jnp.where(best >= FG_T, 1.0, 0.0)
            w0 = jnp.where(best >= IGN_T, fg0f, 1.0)
            lab_old = jnp.where(best >= FG_T, labv, 0)

            mm = jnp.zeros((16,), jnp.int32)
            occ = jnp.full((16,), G, jnp.int32)
            for gp in range(G):
                cond = nj == _splat(bvv[gp // 16], gp % 16)
                mm = jnp.where(cond, gp, mm)
                occ = jnp.minimum(occ, jnp.where(cond, gp, G))
            j_idx = iot + h * 16
            first = occ == j_idx

            ngx1 = _gat(gvv, mm * 4 + 0)
            ngy1 = _gat(gvv, mm * 4 + 1)
            ngx2 = _gat(gvv, mm * 4 + 2)
            ngy2 = _gat(gvv, mm * 4 + 3)
            lab_new = _gat(lvv, mm)

            c0 = gall[pl.ds(0 * G + h * 16, 16)]
            c1 = gall[pl.ds(1 * G + h * 16, 16)]
            c2 = gall[pl.ds(2 * G + h * 16, 16)]
            c3 = gall[pl.ds(3 * G + h * 16, 16)]
            b0 = gall[pl.ds(4 * G + h * 16, 16)]
            b1 = gall[pl.ds(5 * G + h * 16, 16)]
            b2 = gall[pl.ds(6 * G + h * 16, 16)]
            b3 = gall[pl.ds(7 * G + h * 16, 16)]

            mx = jnp.maximum(jnp.maximum(c0, c1), jnp.maximum(c2, c3))
            lse = _sc_log(jnp.exp(c0 - mx) + jnp.exp(c1 - mx)
                          + jnp.exp(c2 - mx) + jnp.exp(c3 - mx)) + mx

            def csel(lab):
                return jnp.where(lab == 0, c0,
                                 jnp.where(lab == 1, c1,
                                           jnp.where(lab == 2, c2, c3)))
            ce_old = lse - csel(lab_old)
            ce_new = lse - csel(lab_new)

            def sl1_of(q1, q2, q3, q4):
                gwv = q3 - q1 + 1.0
                ghv = q4 - q2 + 1.0
                gcx = q1 + 0.5 * gwv
                gcy = q2 + 0.5 * ghv
                t0 = ((gcx - rcx) / rw) / STDS[0]
                t1 = ((gcy - rcy) / rh) / STDS[1]
                t2 = _sc_log(gwv / rw) / STDS[2]
                t3 = _sc_log(ghv / rh) / STDS[3]
                s = jnp.zeros((16,), jnp.float32)
                for bv, tv in ((b0, t0), (b1, t1), (b2, t2), (b3, t3)):
                    d = bv - tv
                    ad = jnp.abs(d)
                    s = s + jnp.where(ad < 1.0, 0.5 * d * d, ad - 0.5)
                return s

            sl1_old = sl1_of(ox1, oy1, ox2, oy2)
            sl1_new = sl1_of(ngx1, ngy1, ngx2, ngy2)

            d0 = b0 * STDS[0]
            d1 = b1 * STDS[1]
            d2 = b2 * STDS[2]
            d3 = b3 * STDS[3]
            pcx = d0 * rw + rcx
            pcy = d1 * rh + rcy
            pw = jnp.exp(jnp.clip(d2, -4.0, 4.0)) * rw
            ph = jnp.exp(jnp.clip(d3, -4.0, 4.0)) * rh
            px1 = pcx - 0.5 * pw
            py1 = pcy - 0.5 * ph
            px2 = pcx + 0.5 * pw
            py2 = pcy + 0.5 * ph
            pa = (px2 - px1) * (py2 - py1)

            def iou_of(q1, q2, q3, q4):
                iw = jnp.maximum(jnp.minimum(px2, q3) - jnp.maximum(px1, q1), 0.0)
                ih = jnp.maximum(jnp.minimum(py2, q4) - jnp.maximum(py1, q2), 0.0)
                inter = iw * ih
                ga = (q3 - q1) * (q4 - q2)
                return inter / jnp.maximum(pa + ga - inter, 1e-8)

            iou_old = iou_of(ox1, oy1, ox2, oy2)
            iou_new = iou_of(ngx1, ngy1, ngx2, ngy2)

            zf = jnp.zeros((16,), jnp.float32)
            dd[0] = dd[0] + jnp.where(first, ce_new - ce_old * w0, zf)
            dd[1] = dd[1] + jnp.where(first, 1.0 - w0, zf)
            dd[2] = dd[2] + jnp.where(first, sl1_new - sl1_old * fg0f, zf)
            dd[3] = dd[3] + jnp.where(first,
                                      (1.0 - iou_new) - (1.0 - iou_old) * fg0f,
                                      zf)
            dd[4] = dd[4] + jnp.where(first, 1.0 - fg0f, zf)

        cls_num = _splat(svv, 0) + _lane_sum(dd[0])
        w_sum = _splat(svv, 1) + _lane_sum(dd[1])
        bb_num = _splat(svv, 2) + _lane_sum(dd[2])
        iou_num = _splat(svv, 3) + _lane_sum(dd[3])
        nfg = _splat(svv, 4) + _lane_sum(dd[4])
        loss = (cls_num / jnp.maximum(w_sum, 1.0)
                + (bb_num + iou_num) / jnp.maximum(nfg, 1.0))
        outs[...] = loss
        pltpu.sync_copy(outs, out_h.at[pl.ds(i * 16, 16)])


def _sc_stage(gtsf, lblf, ancf, sumsf, bindsf, gathf):
    mesh = plsc.VectorSubcoreMesh(core_axis_name="c", subcore_axis_name="s")
    run = functools.partial(
        pl.kernel,
        mesh=mesh,
        out_type=jax.ShapeDtypeStruct((B * 16,), jnp.float32),
        scratch_types=[
            pltpu.VMEM((4 * G,), jnp.float32),
            pltpu.VMEM((G,), jnp.int32),
            pltpu.VMEM((G,), jnp.int32),
            pltpu.VMEM((16,), jnp.float32),
            pltpu.VMEM((48,), jnp.float32),
            pltpu.VMEM((256,), jnp.float32),
            pltpu.VMEM((16,), jnp.float32),
        ],
    )(_sc_body)
    return run(gtsf, lblf, ancf, sumsf, bindsf, gathf)


@jax.jit
def kernel(cls, bbox_2d, gts, anchors, gt_labels):
    cls_t = cls.transpose(0, 2, 1).reshape(B, 4, ROWS, LANES)
    box_t = bbox_2d.transpose(0, 2, 1).reshape(B, 4, ROWS, LANES)
    lbl = gt_labels.astype(jnp.int32)
    sums, binds, gath = _tc_stage(cls_t, box_t, gts, lbl, anchors)
    out = _sc_stage(
        gts.reshape(-1),
        lbl.reshape(-1),
        jnp.pad(anchors.reshape(-1), (0, 12)),
        sums.reshape(-1),
        binds.reshape(-1),
        gath.reshape(-1),
    )
    return jnp.mean(out.reshape(B, 16)[:, 0])
